# baseline (speedup 1.0000x reference)
import functools

import jax
import jax.numpy as jnp
from jax import lax
from jax.experimental import pallas as pl
from jax.experimental.pallas import tpu as pltpu
from jax.experimental.pallas import tpu_sc as plsc

_S_BLK = 2048


def _add_body(x_ref, pe_ref, o_ref):
    o_ref[...] = x_ref[...] + pe_ref[...][None, :, :]


def _tc_add(x, pe):
    B, S, D = x.shape
    grid = (S // _S_BLK, B)
    return pl.pallas_call(
        _add_body,
        grid=grid,
        in_specs=[
            pl.BlockSpec((1, _S_BLK, D), lambda j, b: (b, j, 0)),
            pl.BlockSpec((_S_BLK, D), lambda j, b: (j, 0)),
        ],
        out_specs=pl.BlockSpec((1, _S_BLK, D), lambda j, b: (b, j, 0)),
        out_shape=jax.ShapeDtypeStruct((B, S, D), x.dtype),
    )(x, pe)


_PE_N = 4 * 1024 * 1024
_NC, _NS = 2, 16
_NW = _NC * _NS
_PC = _PE_N // _NW
_PCH = 16384
_PSTEPS = _PC // _PCH
_PLN = 4


def _sc_probe_body(pe_hbm, out_hbm, b0, b1, b2, b3,
                   l0, l1, l2, l3, s0, s1, s2, s3):
    wid = lax.axis_index("s") * _NC + lax.axis_index("c")
    base = wid * _PC
    bufs = (b0, b1, b2, b3)
    lds = (l0, l1, l2, l3)
    sts = (s0, s1, s2, s3)

    def off(t):
        return pl.multiple_of(base + t * _PCH, _PCH)

    def load(t):
        return pltpu.async_copy(pe_hbm.at[pl.ds(off(t), _PCH)], bufs[t % _PLN], lds[t % _PLN])

    def store(t):
        return pltpu.async_copy(bufs[t % _PLN], out_hbm.at[pl.ds(off(t), _PCH)], sts[t % _PLN])

    ld = [load(0), load(1), load(2), load(3)]
    st = [None, None, None, None]
    for t in range(_PSTEPS):
        k = t % _PLN
        ld[k].wait()
        st[k] = store(t)
        if t + _PLN < _PSTEPS:
            st[k].wait()
            ld[k] = load(t + _PLN)
    for k in range(_PLN):
        if st[k] is not None:
            st[k].wait()


_sc_probe = functools.partial(
    pl.kernel,
    mesh=plsc.VectorSubcoreMesh(core_axis_name="c", subcore_axis_name="s"),
    out_type=jax.ShapeDtypeStruct((_PE_N,), jnp.float32),

    scratch_types=(
        [pltpu.VMEM((_PCH,), jnp.float32)] * 4 + [pltpu.SemaphoreType.DMA] * 8
    ),
)(_sc_probe_body)


def kernel(x, pe):
    y = _tc_add(x, pe)
    z = _sc_probe(pe.reshape(-1))
    patch = y[0, 0, :16] + 0.0 * z[:16]
    return y.at[0, 0, :16].set(patch)


# speedup vs baseline: 2.2225x; 2.2225x over previous
import jax
import jax.numpy as jnp
from jax.experimental import pallas as pl


_S_BLK = 2048


def _add_body(x_ref, pe_ref, o_ref):
    o_ref[...] = x_ref[...] + pe_ref[...][None, :, :]


def kernel(x, pe):
    B, S, D = x.shape
    grid = (S // _S_BLK, B)
    return pl.pallas_call(
        _add_body,
        grid=grid,
        in_specs=[
            pl.BlockSpec((1, _S_BLK, D), lambda j, b: (b, j, 0)),
            pl.BlockSpec((_S_BLK, D), lambda j, b: (j, 0)),
        ],
        out_specs=pl.BlockSpec((1, _S_BLK, D), lambda j, b: (b, j, 0)),
        out_shape=jax.ShapeDtypeStruct((B, S, D), x.dtype),
    )(x, pe)
